# Initial kernel scaffold; baseline (speedup 1.0000x reference)
import jax
import jax.numpy as jnp
from jax.experimental import pallas as pl


def kernel(prediction, target):
    raise NotImplementedError("write your pallas kernel here")


# speedup vs baseline: 2.4161x; 2.4161x over previous
import functools

import jax
import jax.numpy as jnp
from jax.experimental import pallas as pl
from jax.experimental.pallas import tpu as pltpu

_NC = 20
_LC = 5.0
_LN = 0.5
_A = 5
_ANCHORS = ((1.3221, 1.73145), (3.19275, 4.00944), (5.05587, 8.09892),
            (9.47112, 4.84053), (11.2364, 10.0071))
_SENTINEL = 2 ** 30


def _sig(x):
    return 1.0 / (1.0 + jnp.exp(-x))


def _loss_body(pred_ref, targ_ref, out_ref, s_main, s_cls, s_key, *, B, HW):
    b = pl.program_id(0)

    @pl.when(b == 0)
    def _init():
        s_main[0] = 0.0
        for a in range(_A):
            s_cls[a] = 0.0
        s_key[0] = _SENTINEL

    p = pred_ref[0]
    t = targ_ref[0]

    gcls = t[0:_NC, :]
    conf_t = t[_NC:_NC + 1, :]
    gxy = t[_NC + 1:_NC + 3, :]
    gwh = t[_NC + 3:_NC + 5, :]
    obj = (conf_t != 0.0)
    obj_f = obj.astype(jnp.float32)

    b_min = gxy - gwh * 0.5
    b_max = gxy + gwh * 0.5
    area_b = gwh[0:1, :] * gwh[1:2, :]

    xys = []
    whs = []
    confs = []
    clss = []
    best_iou = None
    best_a = None
    for a in range(_A):
        base = a * (5 + _NC)
        cls_a = p[base:base + _NC, :]
        conf_a = _sig(p[base + _NC:base + _NC + 1, :])
        xy_a = _sig(p[base + _NC + 1:base + _NC + 3, :])
        wh_a = jnp.exp(p[base + _NC + 3:base + _NC + 5, :])
        aw, ah = _ANCHORS[a]
        anc = wh_a * jnp.concatenate(
            [jnp.full((1, wh_a.shape[1]), aw, jnp.float32),
             jnp.full((1, wh_a.shape[1]), ah, jnp.float32)], axis=0)
        a_min = xy_a - anc * 0.5
        a_max = xy_a + anc * 0.5
        lt = jnp.maximum(a_min, b_min)
        rb = jnp.minimum(a_max, b_max)
        iw = jnp.clip(rb - lt, 0.0, None)
        inter = iw[0:1, :] * iw[1:2, :]
        area_a = anc[0:1, :] * anc[1:2, :]
        iou = inter / (area_a + area_b - inter + 1e-9)
        xys.append(xy_a)
        whs.append(wh_a)
        confs.append(conf_a)
        clss.append(cls_a)
        if a == 0:
            best_iou = iou
            best_a = jnp.zeros(iou.shape, jnp.int32)
        else:
            gt = iou > best_iou
            best_iou = jnp.where(gt, iou, best_iou)
            best_a = jnp.where(gt, jnp.int32(a), best_a)

    box_s = 0.0
    conf_s = 0.0
    noobj_s = 0.0
    for a in range(_A):
        m = obj_f * (best_a == a).astype(jnp.float32)
        dxy = xys[a] - gxy
        dwh = whs[a] - gwh
        box_s = box_s + jnp.sum(m * (dxy * dxy)) + jnp.sum(m * (dwh * dwh))
        cm1 = confs[a] - 1.0
        conf_s = conf_s + jnp.sum(m * cm1 * cm1)
        noobj_s = noobj_s + jnp.sum((1.0 - m) * confs[a] * confs[a])
        mx = jnp.max(clss[a], axis=0, keepdims=True)
        se = jnp.sum(jnp.exp(clss[a] - mx), axis=0, keepdims=True)
        picked = jnp.sum(gcls * clss[a], axis=0, keepdims=True) - mx - jnp.log(se)
        s_cls[a] = s_cls[a] - jnp.sum(obj_f * picked)

    s_main[0] = s_main[0] + _LC * box_s + conf_s + _LN * noobj_s

    lane = jax.lax.broadcasted_iota(jnp.int32, (1, HW), 1)
    flat = b * HW + lane
    key = jnp.where(obj, flat * 8 + best_a, _SENTINEL)
    s_key[0] = jnp.minimum(s_key[0], jnp.min(key))

    @pl.when(b == B - 1)
    def _finish():
        a0 = jax.lax.rem(s_key[0], jnp.int32(8))
        out_ref[0, 0] = s_main[0] + s_cls[a0]


def kernel(prediction, target):
    B, C, H, W = prediction.shape
    HW = H * W
    pred = prediction.reshape(B, C, HW)
    targ = jnp.transpose(target, (0, 2, 1))

    out = pl.pallas_call(
        functools.partial(_loss_body, B=B, HW=HW),
        grid=(B,),
        in_specs=[
            pl.BlockSpec((1, C, HW), lambda b: (b, 0, 0)),
            pl.BlockSpec((1, 5 + _NC, HW), lambda b: (b, 0, 0)),
        ],
        out_specs=pl.BlockSpec(memory_space=pltpu.SMEM),
        out_shape=jax.ShapeDtypeStruct((1, 1), jnp.float32),
        scratch_shapes=[
            pltpu.SMEM((1,), jnp.float32),
            pltpu.SMEM((_A,), jnp.float32),
            pltpu.SMEM((1,), jnp.int32),
        ],
    )(pred, targ)
    return out.reshape(())
